# baseline (speedup 1.0000x reference)
import jax
import jax.numpy as jnp
from jax import lax
from jax.experimental import pallas as pl
from jax.experimental.pallas import tpu as pltpu
from jax.experimental.pallas import tpu_sc as plsc

N_ROWS = 1_000_000
N_COLS = 4
NUM_DIST = 64
NUM_ANGLE = 32

NC, NS, L = 2, 16, 16
NW = NC * NS
B_ROWS = 512
B_FLAT = B_ROWS * N_COLS
N_GATHER = B_FLAT // 128
N_CHUNKS = (N_ROWS + B_ROWS - 1) // B_ROWS
TRIPS = (N_CHUNKS + NW - 1) // NW
LAST_BASE = N_ROWS - B_ROWS

_FIELD_COL = (1, 2, 3, 0)


def _body(emb_hbm, tab_hbm, out_hbm, embc, idx1, rows, semg):
    c = lax.axis_index("c")
    s = lax.axis_index("s")
    wid = s * NC + c
    lane = lax.iota(jnp.int32, L)

    def chunk_body(k, carry):
        i = wid + k * NW

        @pl.when(i < N_CHUNKS)
        def _do():
            base = jnp.minimum(i * B_ROWS, LAST_BASE)
            pltpu.sync_copy(emb_hbm.at[pl.ds(base, B_ROWS)], embc)

            for f in range(4):
                col = _FIELD_COL[f]
                dist = col == 0
                scl = jnp.float32(64.0 if dist else 32.0)
                mx = 63 if dist else 31
                off = 32 if dist else 0

                def vec_body(v, inner, f=f, col=col, scl=scl, mx=mx, off=off):
                    idx1[pl.ds(f * B_ROWS + v * L, L)] = lane + off
                    return inner

                lax.fori_loop(0, B_ROWS // L, vec_body, 0)

            copies = [
                pltpu.async_copy(
                    tab_hbm.at[idx1.at[pl.ds(j * 128, 128)]],
                    rows.at[pl.ds(j * 128, 128)],
                    semg,
                )
                for j in range(N_GATHER)
            ]
            for cp in copies:
                cp.wait()

            for f in range(4):
                pltpu.sync_copy(
                    rows.at[pl.ds(f * B_ROWS, B_ROWS)],
                    out_hbm.at[pl.ds(base, B_ROWS), pl.ds(f * L, L)],
                )

        return carry

    lax.fori_loop(0, TRIPS, chunk_body, 0)


@jax.jit
def _sc_call(emb, table):
    mesh = plsc.VectorSubcoreMesh(
        core_axis_name="c", subcore_axis_name="s", num_cores=NC, num_subcores=NS
    )
    return pl.kernel(
        _body,
        out_type=jax.ShapeDtypeStruct((N_ROWS, 64), jnp.float32),
        mesh=mesh,
        compiler_params=pltpu.CompilerParams(
            needs_layout_passes=False, use_tc_tiling_on_sc=False
        ),
        scratch_types=[
            pltpu.VMEM((B_ROWS, N_COLS), jnp.float32),
            pltpu.VMEM((B_FLAT,), jnp.int32),
            pltpu.VMEM((B_FLAT, 16), jnp.float32),
            pltpu.SemaphoreType.DMA,
        ],
    )(emb, table)


def kernel(embeddings, W_dist, b_dist, W_angle, b_angle):
    table = jnp.concatenate(
        [W_angle + b_angle[None, :], W_dist + b_dist[None, :]], axis=0
    )
    return _sc_call(embeddings, table)


# speedup vs baseline: 3.6310x; 3.6310x over previous
import jax
import jax.numpy as jnp
from jax import lax
from jax.experimental import pallas as pl
from jax.experimental.pallas import tpu as pltpu
from jax.experimental.pallas import tpu_sc as plsc

N_ROWS = 1_000_000
N_COLS = 4
NUM_DIST = 64
NUM_ANGLE = 32

NC, NS, L = 2, 16, 16
NW = NC * NS
B_ROWS = 512
B_FLAT = B_ROWS * N_COLS
N_GATHER = B_FLAT // 128
N_CHUNKS = (N_ROWS + B_ROWS - 1) // B_ROWS
TRIPS = (N_CHUNKS + NW - 1) // NW
LAST_BASE = N_ROWS - B_ROWS

_FIELD_COL = (1, 2, 3, 0)


def _body(emb_hbm, tab_hbm, out_hbm, embc, idx1, rows, tab_sh, semg):
    c = lax.axis_index("c")
    s = lax.axis_index("s")
    wid = s * NC + c
    lane = lax.iota(jnp.int32, L)

    @pl.when(s == 0)
    def _stage():
        pltpu.sync_copy(tab_hbm, tab_sh)

    plsc.subcore_barrier()

    def chunk_body(k, carry):
        i = wid + k * NW

        @pl.when(i < N_CHUNKS)
        def _do():
            base = jnp.minimum(i * B_ROWS, LAST_BASE)
            pltpu.sync_copy(emb_hbm.at[pl.ds(base, B_ROWS)], embc)

            for f in range(4):
                col = _FIELD_COL[f]
                dist = col == 0
                scl = jnp.float32(64.0 if dist else 32.0)
                mx = 63 if dist else 31
                off = 32 if dist else 0

                def vec_body(v, inner, f=f, col=col, scl=scl, mx=mx, off=off):
                    row = lane + v * L
                    cv = jnp.full((L,), col, jnp.int32)
                    e = plsc.load_gather(embc, [row, cv])
                    y = e * scl
                    t = y.astype(jnp.int32)
                    tf = t.astype(jnp.float32)
                    idx = jnp.where(y > tf, t, t - 1)
                    idx = jnp.minimum(jnp.maximum(idx, 0), mx) + off
                    idx1[pl.ds(f * B_ROWS + v * L, L)] = idx
                    return inner

                lax.fori_loop(0, B_ROWS // L, vec_body, 0)

            copies = [
                pltpu.async_copy(
                    tab_sh.at[idx1.at[pl.ds(j * 128, 128)]],
                    rows.at[pl.ds(j * 128, 128)],
                    semg,
                )
                for j in range(N_GATHER)
            ]
            for cp in copies:
                cp.wait()

            for f in range(4):
                pltpu.sync_copy(
                    rows.at[pl.ds(f * B_ROWS, B_ROWS)],
                    out_hbm.at[pl.ds(base, B_ROWS), pl.ds(f * L, L)],
                )

        return carry

    lax.fori_loop(0, TRIPS, chunk_body, 0)


@jax.jit
def _sc_call(emb, table):
    mesh = plsc.VectorSubcoreMesh(
        core_axis_name="c", subcore_axis_name="s", num_cores=NC, num_subcores=NS
    )
    return pl.kernel(
        _body,
        out_type=jax.ShapeDtypeStruct((N_ROWS, 64), jnp.float32),
        mesh=mesh,
        compiler_params=pltpu.CompilerParams(
            needs_layout_passes=False, use_tc_tiling_on_sc=False
        ),
        scratch_types=[
            pltpu.VMEM((B_ROWS, N_COLS), jnp.float32),
            pltpu.VMEM((B_FLAT,), jnp.int32),
            pltpu.VMEM((B_FLAT, 16), jnp.float32),
            pltpu.VMEM_SHARED((96, 16), jnp.float32),
            pltpu.SemaphoreType.DMA,
        ],
    )(emb, table)


def kernel(embeddings, W_dist, b_dist, W_angle, b_angle):
    table = jnp.concatenate(
        [W_angle + b_angle[None, :], W_dist + b_dist[None, :]], axis=0
    )
    return _sc_call(embeddings, table)


# speedup vs baseline: 13.8480x; 3.8138x over previous
import jax
import jax.numpy as jnp
from jax import lax
from jax.experimental import pallas as pl
from jax.experimental.pallas import tpu as pltpu
from jax.experimental.pallas import tpu_sc as plsc

N_ROWS = 1_000_000
NC, NS, L = 2, 16, 16
NW = NC * NS
B_ROWS = 512
N_TILES_R = B_ROWS // 128
N_FULL = N_ROWS // B_ROWS
TAIL_BASE = N_FULL * B_ROWS
TRIPS = (N_FULL + NW - 1) // NW
TRIPS2 = (TRIPS + 1) // 2


def _compute_idx(ecv, idx1):
    for f in range(4):
        dist = f == 3
        scl = jnp.float32(64.0 if dist else 32.0)
        mx = 63 if dist else 31
        off = 32 if dist else 0

        def vec_body(v, inner, f=f, scl=scl, mx=mx, off=off):
            e = ecv[pl.ds(f * B_ROWS + v * L, L)]
            y = e * scl
            t = y.astype(jnp.int32)
            tf = t.astype(jnp.float32)
            idx = jnp.where(y > tf, t, t - 1)
            idx = jnp.minimum(jnp.maximum(idx, 0), mx) + off
            idx1[pl.ds(f * B_ROWS + v * L, L)] = idx
            return inner

        lax.fori_loop(0, B_ROWS // L, vec_body, 0, unroll=4)


def _fill_tiles(idx1, tabv, out3):
    for t in range(N_TILES_R):

        def row_body(w, inner, t=t):
            r0 = t * 128 + w * L
            ivs = tuple(idx1[pl.ds(f * B_ROWS + r0, L)] for f in range(4))
            for g in range(8):
                for u in range(8):
                    feat = 8 * g + u
                    lcol = feat % 16
                    val = plsc.load_gather(
                        tabv.at[pl.ds(lcol * 96, 96)], [ivs[feat // 16]]
                    )
                    out3[4 * g + t, u, pl.ds(w * L, L)] = val
            return inner

        lax.fori_loop(0, 128 // L, row_body, 0)


def _in_copies(cols, i, ecv, sem):
    base = pl.multiple_of(i * B_ROWS, B_ROWS)
    return [
        pltpu.make_async_copy(
            cols[f].at[pl.ds(base, B_ROWS)],
            ecv.at[pl.ds(f * B_ROWS, B_ROWS)],
            sem,
        )
        for f in range(4)
    ]


def _out_copies(out_hbm, i, out3, sem):
    base = pl.multiple_of(i * B_ROWS, B_ROWS)
    return [
        pltpu.make_async_copy(
            out3.at[4 * g + t],
            out_hbm.at[pl.ds(8 * g, 8), pl.ds(base + 128 * t, 128)],
            sem,
        )
        for g in range(8)
        for t in range(N_TILES_R)
    ]


def _body(
    e0, e1, e2, e3, tab_hbm, out_hbm,
    ecvA, ecvB, idx1, tabv, out3A, out3B,
    insemA, insemB, outsemA, outsemB,
):
    c = lax.axis_index("c")
    s = lax.axis_index("s")
    wid = s * NC + c
    cols = (e1, e2, e3, e0)

    pltpu.sync_copy(tab_hbm, tabv)

    bufs = (
        (ecvA, out3A, insemA, outsemA),
        (ecvB, out3B, insemB, outsemB),
    )

    for cp in _in_copies(cols, wid, ecvA, insemA):
        cp.start()

    def do_chunk(i, m, p):
        ecv, out3, insem, outsem = bufs[p]
        nxt_i = i + NW

        @pl.when(nxt_i < N_FULL)
        def _prefetch():
            ecv2, _, insem2, _ = bufs[1 - p]
            for cp in _in_copies(cols, nxt_i, ecv2, insem2):
                cp.start()

        for cp in _in_copies(cols, i, ecv, insem):
            cp.wait()
        _compute_idx(ecv, idx1)

        @pl.when(m >= 1)
        def _drain_prev():
            for cp in _out_copies(out_hbm, i - 2 * NW, out3, outsem):
                cp.wait()

        _fill_tiles(idx1, tabv, out3)
        for cp in _out_copies(out_hbm, i, out3, outsem):
            cp.start()

    def iter_body(m, carry):
        i0 = wid + (2 * m) * NW
        i1 = i0 + NW

        @pl.when(i0 < N_FULL)
        def _c0():
            do_chunk(i0, m, 0)

        @pl.when(i1 < N_FULL)
        def _c1():
            do_chunk(i1, m, 1)

        return carry

    lax.fori_loop(0, TRIPS2, iter_body, 0)

    kmax = (N_FULL - 1 - wid) // NW
    for p in (0, 1):
        k_p = jnp.where(kmax % 2 == p, kmax, kmax - 1)
        i_p = wid + k_p * NW
        _, out3, _, outsem = bufs[p]
        for cp in _out_copies(out_hbm, i_p, out3, outsem):
            cp.wait()


@jax.jit
def _sc_call(e0, e1, e2, e3, tab_t_flat):
    mesh = plsc.VectorSubcoreMesh(
        core_axis_name="c", subcore_axis_name="s", num_cores=NC, num_subcores=NS
    )
    return pl.kernel(
        _body,
        out_type=jax.ShapeDtypeStruct((64, N_ROWS), jnp.float32),
        mesh=mesh,
        compiler_params=pltpu.CompilerParams(
            needs_layout_passes=False, use_tc_tiling_on_sc=True
        ),
        scratch_types=[
            pltpu.VMEM((4 * B_ROWS,), jnp.float32),
            pltpu.VMEM((4 * B_ROWS,), jnp.float32),
            pltpu.VMEM((4 * B_ROWS,), jnp.int32),
            pltpu.VMEM((16 * 96,), jnp.float32),
            pltpu.VMEM((8 * N_TILES_R, 8, 128), jnp.float32),
            pltpu.VMEM((8 * N_TILES_R, 8, 128), jnp.float32),
            pltpu.SemaphoreType.DMA,
            pltpu.SemaphoreType.DMA,
            pltpu.SemaphoreType.DMA,
            pltpu.SemaphoreType.DMA,
        ],
    )(e0, e1, e2, e3, tab_t_flat)


def _bucket(x, nb):
    y = x * jnp.float32(nb)
    t = y.astype(jnp.int32)
    idx = jnp.where(y > t.astype(jnp.float32), t, t - 1)
    return jnp.clip(idx, 0, nb - 1)


def kernel(embeddings, W_dist, b_dist, W_angle, b_angle):
    table = jnp.concatenate(
        [W_angle + b_angle[None, :], W_dist + b_dist[None, :]], axis=0
    )
    out_t = _sc_call(
        embeddings[:, 0],
        embeddings[:, 1],
        embeddings[:, 2],
        embeddings[:, 3],
        table.T.reshape(-1),
    )
    out = out_t.T
    te = embeddings[TAIL_BASE:]
    tvals = jnp.concatenate(
        [
            table[_bucket(te[:, 1], 32)],
            table[_bucket(te[:, 2], 32)],
            table[_bucket(te[:, 3], 32)],
            table[32 + _bucket(te[:, 0], 64)],
        ],
        axis=1,
    )
    return lax.dynamic_update_slice(out, tvals, (TAIL_BASE, 0))


# speedup vs baseline: 15.1554x; 1.0944x over previous
import jax
import jax.numpy as jnp
from jax import lax
from jax.experimental import pallas as pl
from jax.experimental.pallas import tpu as pltpu
from jax.experimental.pallas import tpu_sc as plsc

N_ROWS = 1_000_000
NC, NS, L = 2, 16, 16
NW = NC * NS
B_ROWS = 512
N_TILES_R = B_ROWS // 128
N_FULL = N_ROWS // B_ROWS
TAIL_BASE = N_FULL * B_ROWS
TRIPS = (N_FULL + NW - 1) // NW
TRIPS2 = (TRIPS + 1) // 2


def _compute_idx(ecv, idx1):
    for f in range(4):
        dist = f == 3
        scl = jnp.float32(64.0 if dist else 32.0)
        mx = 63 if dist else 31
        off = 32 if dist else 0

        def vec_body(v, inner, f=f, scl=scl, mx=mx, off=off):
            e = ecv[pl.ds(f * B_ROWS + v * L, L)]
            y = e * scl
            t = y.astype(jnp.int32)
            tf = t.astype(jnp.float32)
            idx = jnp.where(y > tf, t, t - 1)
            idx = jnp.minimum(jnp.maximum(idx, 0), mx) + off
            idx1[pl.ds(f * B_ROWS + v * L, L)] = idx
            return inner

        lax.fori_loop(0, B_ROWS // L, vec_body, 0, unroll=4)


def _fill_tiles(idx1, tabv, out3):
    def row_body(w, inner):
        r0 = w * L
        ivs = tuple(idx1[pl.ds(f * B_ROWS + r0, L)] for f in range(4))
        for feat in range(64):
            lcol = feat % 16
            val = plsc.load_gather(
                tabv.at[pl.ds(lcol * 96, 96)], [ivs[feat // 16]]
            )
            out3[feat, pl.ds(r0, L)] = val
        return inner

    lax.fori_loop(0, B_ROWS // L, row_body, 0)


def _in_copies(cols, i, ecv, sem):
    base = pl.multiple_of(i * B_ROWS, B_ROWS)
    return [
        pltpu.make_async_copy(
            cols[f].at[pl.ds(base, B_ROWS)],
            ecv.at[pl.ds(f * B_ROWS, B_ROWS)],
            sem,
        )
        for f in range(4)
    ]


def _out_copies(out_hbm, i, out3, sem):
    base = pl.multiple_of(i * B_ROWS, B_ROWS)
    return [
        pltpu.make_async_copy(
            out3,
            out_hbm.at[:, pl.ds(base, B_ROWS)],
            sem,
        )
    ]


def _body(
    e0, e1, e2, e3, tab_hbm, out_hbm,
    ecvA, ecvB, idx1, tabv, out3A, out3B,
    insemA, insemB, outsemA, outsemB,
):
    c = lax.axis_index("c")
    s = lax.axis_index("s")
    wid = s * NC + c
    cols = (e1, e2, e3, e0)

    pltpu.sync_copy(tab_hbm, tabv)

    bufs = (
        (ecvA, out3A, insemA, outsemA),
        (ecvB, out3B, insemB, outsemB),
    )

    for cp in _in_copies(cols, wid, ecvA, insemA):
        cp.start()

    def do_chunk(i, m, p):
        ecv, out3, insem, outsem = bufs[p]
        nxt_i = i + NW

        @pl.when(nxt_i < N_FULL)
        def _prefetch():
            ecv2, _, insem2, _ = bufs[1 - p]
            for cp in _in_copies(cols, nxt_i, ecv2, insem2):
                cp.start()

        for cp in _in_copies(cols, i, ecv, insem):
            cp.wait()
        _compute_idx(ecv, idx1)

        @pl.when(m >= 1)
        def _drain_prev():
            for cp in _out_copies(out_hbm, i - 2 * NW, out3, outsem):
                cp.wait()

        _fill_tiles(idx1, tabv, out3)
        for cp in _out_copies(out_hbm, i, out3, outsem):
            cp.start()

    def iter_body(m, carry):
        i0 = wid + (2 * m) * NW
        i1 = i0 + NW

        @pl.when(i0 < N_FULL)
        def _c0():
            do_chunk(i0, m, 0)

        @pl.when(i1 < N_FULL)
        def _c1():
            do_chunk(i1, m, 1)

        return carry

    lax.fori_loop(0, TRIPS2, iter_body, 0)

    kmax = (N_FULL - 1 - wid) // NW
    for p in (0, 1):
        k_p = jnp.where(kmax % 2 == p, kmax, kmax - 1)
        i_p = wid + k_p * NW
        _, out3, _, outsem = bufs[p]
        for cp in _out_copies(out_hbm, i_p, out3, outsem):
            cp.wait()


@jax.jit
def _sc_call(e0, e1, e2, e3, tab_t_flat):
    mesh = plsc.VectorSubcoreMesh(
        core_axis_name="c", subcore_axis_name="s", num_cores=NC, num_subcores=NS
    )
    return pl.kernel(
        _body,
        out_type=jax.ShapeDtypeStruct((64, N_ROWS), jnp.float32),
        mesh=mesh,
        compiler_params=pltpu.CompilerParams(
            needs_layout_passes=False, use_tc_tiling_on_sc=True
        ),
        scratch_types=[
            pltpu.VMEM((4 * B_ROWS,), jnp.float32),
            pltpu.VMEM((4 * B_ROWS,), jnp.float32),
            pltpu.VMEM((4 * B_ROWS,), jnp.int32),
            pltpu.VMEM((16 * 96,), jnp.float32),
            pltpu.VMEM((64, B_ROWS), jnp.float32),
            pltpu.VMEM((64, B_ROWS), jnp.float32),
            pltpu.SemaphoreType.DMA,
            pltpu.SemaphoreType.DMA,
            pltpu.SemaphoreType.DMA,
            pltpu.SemaphoreType.DMA,
        ],
    )(e0, e1, e2, e3, tab_t_flat)


def _bucket(x, nb):
    y = x * jnp.float32(nb)
    t = y.astype(jnp.int32)
    idx = jnp.where(y > t.astype(jnp.float32), t, t - 1)
    return jnp.clip(idx, 0, nb - 1)


def kernel(embeddings, W_dist, b_dist, W_angle, b_angle):
    table = jnp.concatenate(
        [W_angle + b_angle[None, :], W_dist + b_dist[None, :]], axis=0
    )
    out_t = _sc_call(
        embeddings[:, 0],
        embeddings[:, 1],
        embeddings[:, 2],
        embeddings[:, 3],
        table.T.reshape(-1),
    )
    out = out_t.T
    te = embeddings[TAIL_BASE:]
    tvals = jnp.concatenate(
        [
            table[_bucket(te[:, 1], 32)],
            table[_bucket(te[:, 2], 32)],
            table[_bucket(te[:, 3], 32)],
            table[32 + _bucket(te[:, 0], 64)],
        ],
        axis=1,
    )
    return lax.dynamic_update_slice(out, tvals, (TAIL_BASE, 0))


# speedup vs baseline: 16.7139x; 1.1028x over previous
import jax
import jax.numpy as jnp
from jax import lax
from jax.experimental import pallas as pl
from jax.experimental.pallas import tpu as pltpu
from jax.experimental.pallas import tpu_sc as plsc

N_ROWS = 1_000_000
NC, NS, L = 2, 16, 16
NW = NC * NS
B_ROWS = 512
N_TILES_R = B_ROWS // 128
N_FULL = N_ROWS // B_ROWS
TAIL_BASE = N_FULL * B_ROWS
TRIPS = (N_FULL + NW - 1) // NW
TRIPS2 = (TRIPS + 1) // 2


def _fill_tiles(ecv, tabv, out3):
    def row_body(w, inner):
        r0 = w * L
        ivs = []
        for f in range(4):
            dist = f == 3
            scl = jnp.float32(64.0 if dist else 32.0)
            mx = 63 if dist else 31
            e = ecv[pl.ds(f * B_ROWS + r0, L)]
            y = e * scl
            t = y.astype(jnp.int32)
            tf = t.astype(jnp.float32)
            idx = jnp.where(y > tf, t, t - 1)
            ivs.append(jnp.minimum(jnp.maximum(idx, 0), mx))
        for feat in range(64):
            f = feat // 16
            lcol = feat % 16
            tbase = lcol * 96 + (32 if f == 3 else 0)
            tlen = 64 if f == 3 else 32
            val = plsc.load_gather(tabv.at[pl.ds(tbase, tlen)], [ivs[f]])
            out3[feat, pl.ds(r0, L)] = val
        return inner

    lax.fori_loop(0, B_ROWS // L, row_body, 0)


def _in_copies(cols, i, ecv, sem):
    base = pl.multiple_of(i * B_ROWS, B_ROWS)
    return [
        pltpu.make_async_copy(
            cols[f].at[pl.ds(base, B_ROWS)],
            ecv.at[pl.ds(f * B_ROWS, B_ROWS)],
            sem,
        )
        for f in range(4)
    ]


def _out_copies(out_hbm, i, out3, sem):
    base = pl.multiple_of(i * B_ROWS, B_ROWS)
    return [
        pltpu.make_async_copy(
            out3,
            out_hbm.at[:, pl.ds(base, B_ROWS)],
            sem,
        )
    ]


def _body(
    e0, e1, e2, e3, tab_hbm, out_hbm,
    ecvA, ecvB, tabv, out3A, out3B,
    insemA, insemB, outsemA, outsemB,
):
    c = lax.axis_index("c")
    s = lax.axis_index("s")
    wid = s * NC + c
    cols = (e1, e2, e3, e0)

    pltpu.sync_copy(tab_hbm, tabv)

    bufs = (
        (ecvA, out3A, insemA, outsemA),
        (ecvB, out3B, insemB, outsemB),
    )

    for cp in _in_copies(cols, wid, ecvA, insemA):
        cp.start()

    def do_chunk(i, m, p):
        ecv, out3, insem, outsem = bufs[p]
        nxt_i = i + NW

        @pl.when(nxt_i < N_FULL)
        def _prefetch():
            ecv2, _, insem2, _ = bufs[1 - p]
            for cp in _in_copies(cols, nxt_i, ecv2, insem2):
                cp.start()

        for cp in _in_copies(cols, i, ecv, insem):
            cp.wait()

        @pl.when(m >= 1)
        def _drain_prev():
            for cp in _out_copies(out_hbm, i - 2 * NW, out3, outsem):
                cp.wait()

        _fill_tiles(ecv, tabv, out3)
        for cp in _out_copies(out_hbm, i, out3, outsem):
            cp.start()

    def iter_body(m, carry):
        i0 = wid + (2 * m) * NW
        i1 = i0 + NW

        @pl.when(i0 < N_FULL)
        def _c0():
            do_chunk(i0, m, 0)

        @pl.when(i1 < N_FULL)
        def _c1():
            do_chunk(i1, m, 1)

        return carry

    lax.fori_loop(0, TRIPS2, iter_body, 0)

    kmax = (N_FULL - 1 - wid) // NW
    for p in (0, 1):
        k_p = jnp.where(kmax % 2 == p, kmax, kmax - 1)
        i_p = wid + k_p * NW
        _, out3, _, outsem = bufs[p]
        for cp in _out_copies(out_hbm, i_p, out3, outsem):
            cp.wait()


@jax.jit
def _sc_call(e0, e1, e2, e3, tab_t_flat):
    mesh = plsc.VectorSubcoreMesh(
        core_axis_name="c", subcore_axis_name="s", num_cores=NC, num_subcores=NS
    )
    return pl.kernel(
        _body,
        out_type=jax.ShapeDtypeStruct((64, N_ROWS), jnp.float32),
        mesh=mesh,
        compiler_params=pltpu.CompilerParams(
            needs_layout_passes=False, use_tc_tiling_on_sc=True
        ),
        scratch_types=[
            pltpu.VMEM((4 * B_ROWS,), jnp.float32),
            pltpu.VMEM((4 * B_ROWS,), jnp.float32),
            pltpu.VMEM((16 * 96,), jnp.float32),
            pltpu.VMEM((64, B_ROWS), jnp.float32),
            pltpu.VMEM((64, B_ROWS), jnp.float32),
            pltpu.SemaphoreType.DMA,
            pltpu.SemaphoreType.DMA,
            pltpu.SemaphoreType.DMA,
            pltpu.SemaphoreType.DMA,
        ],
    )(e0, e1, e2, e3, tab_t_flat)


def _bucket(x, nb):
    y = x * jnp.float32(nb)
    t = y.astype(jnp.int32)
    idx = jnp.where(y > t.astype(jnp.float32), t, t - 1)
    return jnp.clip(idx, 0, nb - 1)


def kernel(embeddings, W_dist, b_dist, W_angle, b_angle):
    table = jnp.concatenate(
        [W_angle + b_angle[None, :], W_dist + b_dist[None, :]], axis=0
    )
    out_t = _sc_call(
        embeddings[:, 0],
        embeddings[:, 1],
        embeddings[:, 2],
        embeddings[:, 3],
        table.T.reshape(-1),
    )
    out = out_t.T
    te = embeddings[TAIL_BASE:]
    tvals = jnp.concatenate(
        [
            table[_bucket(te[:, 1], 32)],
            table[_bucket(te[:, 2], 32)],
            table[_bucket(te[:, 3], 32)],
            table[32 + _bucket(te[:, 0], 64)],
        ],
        axis=1,
    )
    return lax.dynamic_update_slice(out, tvals, (TAIL_BASE, 0))


# speedup vs baseline: 16.7520x; 1.0023x over previous
import jax
import jax.numpy as jnp
from jax import lax
from jax.experimental import pallas as pl
from jax.experimental.pallas import tpu as pltpu
from jax.experimental.pallas import tpu_sc as plsc

N_ROWS = 1_000_000
NC, NS, L = 2, 16, 16
NW = NC * NS
B_ROWS = 512
N_TILES_R = B_ROWS // 128
N_FULL = N_ROWS // B_ROWS
TAIL_BASE = N_FULL * B_ROWS
TRIPS = (N_FULL + NW - 1) // NW
TRIPS2 = (TRIPS + 1) // 2


def _fill_tiles(ecv, tabv, out3):
    def row_body(w, inner):
        r0 = w * L
        ivs = []
        for f in range(4):
            dist = f == 3
            scl = jnp.float32(64.0 if dist else 32.0)
            mx = 63 if dist else 31
            e = ecv[pl.ds(f * B_ROWS + r0, L)]
            y = e * scl
            t = y.astype(jnp.int32)
            tf = t.astype(jnp.float32)
            idx = jnp.where(y > tf, t, t - 1)
            ivs.append(jnp.minimum(jnp.maximum(idx, 0), mx))
        for feat in range(64):
            f = feat // 16
            lcol = feat % 16
            tbase = lcol * 96 + (32 if f == 3 else 0)
            tlen = 64 if f == 3 else 32
            val = plsc.load_gather(tabv.at[pl.ds(tbase, tlen)], [ivs[f]])
            out3[feat, pl.ds(r0, L)] = val
        return inner

    lax.fori_loop(0, B_ROWS // L, row_body, 0, unroll=2)


def _in_copies(cols, i, ecv, sem):
    base = pl.multiple_of(i * B_ROWS, B_ROWS)
    return [
        pltpu.make_async_copy(
            cols[f].at[pl.ds(base, B_ROWS)],
            ecv.at[pl.ds(f * B_ROWS, B_ROWS)],
            sem,
        )
        for f in range(4)
    ]


def _out_copies(out_hbm, i, out3, sem):
    base = pl.multiple_of(i * B_ROWS, B_ROWS)
    return [
        pltpu.make_async_copy(
            out3,
            out_hbm.at[:, pl.ds(base, B_ROWS)],
            sem,
        )
    ]


def _body(
    e0, e1, e2, e3, tab_hbm, out_hbm,
    ecvA, ecvB, tabv, out3A, out3B,
    insemA, insemB, outsemA, outsemB,
):
    c = lax.axis_index("c")
    s = lax.axis_index("s")
    wid = s * NC + c
    cols = (e1, e2, e3, e0)

    pltpu.sync_copy(tab_hbm, tabv)

    bufs = (
        (ecvA, out3A, insemA, outsemA),
        (ecvB, out3B, insemB, outsemB),
    )

    for cp in _in_copies(cols, wid, ecvA, insemA):
        cp.start()

    def do_chunk(i, m, p):
        ecv, out3, insem, outsem = bufs[p]
        nxt_i = i + NW

        @pl.when(nxt_i < N_FULL)
        def _prefetch():
            ecv2, _, insem2, _ = bufs[1 - p]
            for cp in _in_copies(cols, nxt_i, ecv2, insem2):
                cp.start()

        for cp in _in_copies(cols, i, ecv, insem):
            cp.wait()

        @pl.when(m >= 1)
        def _drain_prev():
            for cp in _out_copies(out_hbm, i - 2 * NW, out3, outsem):
                cp.wait()

        _fill_tiles(ecv, tabv, out3)
        for cp in _out_copies(out_hbm, i, out3, outsem):
            cp.start()

    def iter_body(m, carry):
        i0 = wid + (2 * m) * NW
        i1 = i0 + NW

        @pl.when(i0 < N_FULL)
        def _c0():
            do_chunk(i0, m, 0)

        @pl.when(i1 < N_FULL)
        def _c1():
            do_chunk(i1, m, 1)

        return carry

    lax.fori_loop(0, TRIPS2, iter_body, 0)

    kmax = (N_FULL - 1 - wid) // NW
    for p in (0, 1):
        k_p = jnp.where(kmax % 2 == p, kmax, kmax - 1)
        i_p = wid + k_p * NW
        _, out3, _, outsem = bufs[p]
        for cp in _out_copies(out_hbm, i_p, out3, outsem):
            cp.wait()


@jax.jit
def _sc_call(e0, e1, e2, e3, tab_t_flat):
    mesh = plsc.VectorSubcoreMesh(
        core_axis_name="c", subcore_axis_name="s", num_cores=NC, num_subcores=NS
    )
    return pl.kernel(
        _body,
        out_type=jax.ShapeDtypeStruct((64, N_ROWS), jnp.float32),
        mesh=mesh,
        compiler_params=pltpu.CompilerParams(
            needs_layout_passes=False, use_tc_tiling_on_sc=True
        ),
        scratch_types=[
            pltpu.VMEM((4 * B_ROWS,), jnp.float32),
            pltpu.VMEM((4 * B_ROWS,), jnp.float32),
            pltpu.VMEM((16 * 96,), jnp.float32),
            pltpu.VMEM((64, B_ROWS), jnp.float32),
            pltpu.VMEM((64, B_ROWS), jnp.float32),
            pltpu.SemaphoreType.DMA,
            pltpu.SemaphoreType.DMA,
            pltpu.SemaphoreType.DMA,
            pltpu.SemaphoreType.DMA,
        ],
    )(e0, e1, e2, e3, tab_t_flat)


def _bucket(x, nb):
    y = x * jnp.float32(nb)
    t = y.astype(jnp.int32)
    idx = jnp.where(y > t.astype(jnp.float32), t, t - 1)
    return jnp.clip(idx, 0, nb - 1)


def kernel(embeddings, W_dist, b_dist, W_angle, b_angle):
    table = jnp.concatenate(
        [W_angle + b_angle[None, :], W_dist + b_dist[None, :]], axis=0
    )
    out_t = _sc_call(
        embeddings[:, 0],
        embeddings[:, 1],
        embeddings[:, 2],
        embeddings[:, 3],
        table.T.reshape(-1),
    )
    out = out_t.T
    te = embeddings[TAIL_BASE:]
    tvals = jnp.concatenate(
        [
            table[_bucket(te[:, 1], 32)],
            table[_bucket(te[:, 2], 32)],
            table[_bucket(te[:, 3], 32)],
            table[32 + _bucket(te[:, 0], 64)],
        ],
        axis=1,
    )
    return lax.dynamic_update_slice(out, tvals, (TAIL_BASE, 0))
